# Initial kernel scaffold; baseline (speedup 1.0000x reference)
import jax
import jax.numpy as jnp
from jax.experimental import pallas as pl


def kernel(inputs, input_params, sum_params, root_params):
    raise NotImplementedError("write your pallas kernel here")


# speedup vs baseline: 129.4393x; 129.4393x over previous
import functools
import math

import jax
import jax.numpy as jnp
from jax.experimental import pallas as pl
from jax.experimental.pallas import tpu as pltpu


def _body(x_ref, ip_ref, w_ref, rp_ref, o_ref, cur_ref, *, V, K, C, Bt, L):

    VC = 16
    def gather_chunk(ci, _):
        v0 = ci * VC
        xc = x_ref[pl.ds(v0, VC), :]
        ipc = ip_ref[pl.ds(v0, VC)]
        m = jnp.max(ipc, axis=-1, keepdims=True)
        lse = jnp.log(jnp.sum(jnp.exp(ipc - m), axis=-1, keepdims=True)) + m
        xb = xc[:, None, :]
        acc = jnp.broadcast_to(ipc[:, :, 0][:, :, None], (VC, K, Bt))
        for c in range(1, C):
            acc = jnp.where(xb == c, ipc[:, :, c][:, :, None], acc)
        cur_ref[pl.ds(v0, VC)] = acc - lse
        return 0
    jax.lax.fori_loop(0, V // VC, gather_chunk, 0)

    RC = 16
    off = 0
    R = V
    for _ in range(L):
        Rn = R // 2
        rc = min(RC, Rn)
        nch = Rn // rc

        def layer_chunk(ci, _, off=off, rc=rc):
            r0 = ci * rc
            p = cur_ref[pl.ds(2 * r0, 2 * rc)].reshape(rc, 2, K, Bt)
            left = p[:, 0]
            right = p[:, 1]
            ml = jnp.max(left, axis=1, keepdims=True)
            mr = jnp.max(right, axis=1, keepdims=True)
            el = jnp.exp(left - ml)
            er = jnp.exp(right - mr)

            w = w_ref[pl.ds(off + r0, rc)]
            wm = jnp.max(w, axis=-1, keepdims=True)
            wl = jnp.log(jnp.sum(jnp.exp(w - wm), axis=-1, keepdims=True)) + wm
            Wn = jnp.exp(w - wl)

            acc = None
            for i in range(K):
                t = None
                for j in range(K):
                    term = Wn[:, :, i * K + j][:, :, None] * er[:, j, :][:, None, :]
                    t = term if t is None else t + term
                contrib = el[:, i, :][:, None, :] * t
                acc = contrib if acc is None else acc + contrib
            cur_ref[pl.ds(r0, rc)] = jnp.log(acc + 1e-38) + ml + mr
            return 0

        jax.lax.fori_loop(0, nch, layer_chunk, 0)
        off += Rn
        R = Rn

    rp = rp_ref[...]
    rm = jnp.max(rp, axis=0, keepdims=True)
    rl = jnp.log(jnp.sum(jnp.exp(rp - rm), axis=0, keepdims=True)) + rm
    z = cur_ref[0] + (rp - rl)
    zm = jnp.max(z, axis=0, keepdims=True)
    lls = jnp.log(jnp.sum(jnp.exp(z - zm), axis=0, keepdims=True)) + zm
    o_ref[...] = lls[None]


def kernel(inputs, input_params, sum_params, root_params):
    B, V = inputs.shape
    _, K, C = input_params.shape
    L = int(math.log2(V))
    Bt = 256
    G = B // Bt

    xT = inputs.T
    rpb = jnp.broadcast_to(root_params[:, None], (K, B))

    body = functools.partial(_body, V=V, K=K, C=C, Bt=Bt, L=L)
    out = pl.pallas_call(
        body,
        grid=(G,),
        in_specs=[
            pl.BlockSpec((V, Bt), lambda g: (0, g)),
            pl.BlockSpec((V, K, C), lambda g: (0, 0, 0)),
            pl.BlockSpec((V - 1, K, K * K), lambda g: (0, 0, 0)),
            pl.BlockSpec((K, Bt), lambda g: (0, g)),
        ],
        out_specs=pl.BlockSpec((1, 1, Bt), lambda g: (g, 0, 0)),
        out_shape=jax.ShapeDtypeStruct((G, 1, Bt), jnp.float32),
        scratch_shapes=[pltpu.VMEM((V, K, Bt), jnp.float32)],
        compiler_params=pltpu.CompilerParams(
            dimension_semantics=("arbitrary",),
        ),
    )(xT, input_params, sum_params, rpb)
    return out.reshape(B, 1)


# speedup vs baseline: 195.5645x; 1.5109x over previous
import functools
import math

import jax
import jax.numpy as jnp
from jax import lax
from jax.experimental import pallas as pl
from jax.experimental.pallas import tpu as pltpu

_GRP = 16


def _prep_body(w_ref, o_ref, *, K, C2):
    w = w_ref[...]
    wm = jnp.max(w, axis=-1, keepdims=True)
    wl = jnp.log(jnp.sum(jnp.exp(w - wm), axis=-1, keepdims=True)) + wm
    wn = jnp.exp(w - wl).reshape(_GRP * K, C2)
    tiled = jnp.concatenate([wn] * _GRP, axis=1)
    rows = lax.broadcasted_iota(jnp.int32, (_GRP * K, _GRP * C2), 0) // K
    cols = lax.broadcasted_iota(jnp.int32, (_GRP * K, _GRP * C2), 1) // C2
    blk = jnp.where(rows == cols, tiled, 0.0).astype(jnp.bfloat16)
    o_ref[...] = blk[None]


def _body(x_ref, ip_ref, w_ref, wg_ref, rp_ref, o_ref, cur_ref,
          *, V, K, C, Bt, L):

    VC = 16
    def gather_chunk(ci, _):
        v0 = ci * VC
        xc = x_ref[pl.ds(v0, VC), :]
        ipc = ip_ref[pl.ds(v0, VC)]
        m = jnp.max(ipc, axis=-1, keepdims=True)
        lse = jnp.log(jnp.sum(jnp.exp(ipc - m), axis=-1, keepdims=True)) + m
        xb = xc[:, None, :]
        acc = jnp.broadcast_to(ipc[:, :, 0][:, :, None], (VC, K, Bt))
        for c in range(1, C):
            acc = jnp.where(xb == c, ipc[:, :, c][:, :, None], acc)
        cur_ref[pl.ds(v0, VC)] = acc - lse
        return 0
    jax.lax.fori_loop(0, V // VC, gather_chunk, 0)

    R = V
    goff = 0
    for _ in range(L):
        Rn = R // 2
        if Rn < _GRP:
            break

        def layer_chunk(ci, _, goff=goff):
            r0 = ci * _GRP
            p = cur_ref[pl.ds(2 * r0, 2 * _GRP)].reshape(_GRP, 2, K, Bt)
            left = p[:, 0]
            right = p[:, 1]
            ml = jnp.max(left, axis=1, keepdims=True)
            mr = jnp.max(right, axis=1, keepdims=True)
            el = jnp.exp(left - ml)
            er = jnp.exp(right - mr)
            E = jnp.concatenate(
                [el[:, i, :][:, None, :] * er for i in range(K)], axis=1)
            Eb = E.reshape(_GRP * K * K, Bt).astype(jnp.bfloat16)
            Wb = wg_ref[goff + ci]
            o = lax.dot_general(Wb, Eb, (((1,), (0,)), ((), ())),
                                preferred_element_type=jnp.float32)
            o = o.reshape(_GRP, K, Bt)
            cur_ref[pl.ds(r0, _GRP)] = jnp.log(o + 1e-38) + ml + mr
            return 0

        jax.lax.fori_loop(0, Rn // _GRP, layer_chunk, 0)
        goff += Rn // _GRP
        R = Rn

    off = V - R
    while R > 1:
        Rn = R // 2
        p = cur_ref[pl.ds(0, 2 * Rn)].reshape(Rn, 2, K, Bt)
        left = p[:, 0]
        right = p[:, 1]
        ml = jnp.max(left, axis=1, keepdims=True)
        mr = jnp.max(right, axis=1, keepdims=True)
        el = jnp.exp(left - ml)
        er = jnp.exp(right - mr)
        w = w_ref[pl.ds(off, Rn)]
        wm = jnp.max(w, axis=-1, keepdims=True)
        wl = jnp.log(jnp.sum(jnp.exp(w - wm), axis=-1, keepdims=True)) + wm
        Wn = jnp.exp(w - wl)
        acc = None
        for i in range(K):
            t = None
            for j in range(K):
                term = Wn[:, :, i * K + j][:, :, None] * er[:, j, :][:, None, :]
                t = term if t is None else t + term
            contrib = el[:, i, :][:, None, :] * t
            acc = contrib if acc is None else acc + contrib
        cur_ref[pl.ds(0, Rn)] = jnp.log(acc + 1e-38) + ml + mr
        off += Rn
        R = Rn

    rp = rp_ref[...]
    rm = jnp.max(rp, axis=0, keepdims=True)
    rl = jnp.log(jnp.sum(jnp.exp(rp - rm), axis=0, keepdims=True)) + rm
    z = cur_ref[0] + (rp - rl)
    zm = jnp.max(z, axis=0, keepdims=True)
    lls = jnp.log(jnp.sum(jnp.exp(z - zm), axis=0, keepdims=True)) + zm
    o_ref[...] = lls[None]


def kernel(inputs, input_params, sum_params, root_params):
    B, V = inputs.shape
    _, K, C = input_params.shape
    C2 = K * K
    L = int(math.log2(V))
    Bt = 256
    G = B // Bt
    NG = sum(
        (V >> (l + 1)) // _GRP for l in range(L) if (V >> (l + 1)) >= _GRP)

    prep = functools.partial(_prep_body, K=K, C2=C2)
    wg = pl.pallas_call(
        prep,
        grid=(NG,),
        in_specs=[pl.BlockSpec((_GRP, K, C2), lambda g: (g, 0, 0))],
        out_specs=pl.BlockSpec((1, _GRP * K, _GRP * C2), lambda g: (g, 0, 0)),
        out_shape=jax.ShapeDtypeStruct((NG, _GRP * K, _GRP * C2), jnp.bfloat16),
        compiler_params=pltpu.CompilerParams(
            dimension_semantics=("arbitrary",),
        ),
    )(sum_params)

    xT = inputs.T
    rpb = jnp.broadcast_to(root_params[:, None], (K, B))

    body = functools.partial(_body, V=V, K=K, C=C, Bt=Bt, L=L)
    out = pl.pallas_call(
        body,
        grid=(G,),
        in_specs=[
            pl.BlockSpec((V, Bt), lambda g: (0, g)),
            pl.BlockSpec((V, K, C), lambda g: (0, 0, 0)),
            pl.BlockSpec((V - 1, K, K * K), lambda g: (0, 0, 0)),
            pl.BlockSpec((NG, _GRP * K, _GRP * C2), lambda g: (0, 0, 0)),
            pl.BlockSpec((K, Bt), lambda g: (0, g)),
        ],
        out_specs=pl.BlockSpec((1, 1, Bt), lambda g: (g, 0, 0)),
        out_shape=jax.ShapeDtypeStruct((G, 1, Bt), jnp.float32),
        scratch_shapes=[pltpu.VMEM((V, K, Bt), jnp.float32)],
        compiler_params=pltpu.CompilerParams(
            dimension_semantics=("arbitrary",),
        ),
    )(xT, input_params, sum_params, wg, rpb)
    return out.reshape(B, 1)


# speedup vs baseline: 420.9579x; 2.1525x over previous
import functools
import math

import jax
import jax.numpy as jnp
from jax import lax
from jax.experimental import pallas as pl
from jax.experimental.pallas import tpu as pltpu

_GRP = 16


def _blockdiag(wn, K, C2):
    tiled = jnp.concatenate([wn] * _GRP, axis=1)
    rows = lax.broadcasted_iota(jnp.int32, (_GRP * K, _GRP * C2), 0) // K
    cols = lax.broadcasted_iota(jnp.int32, (_GRP * K, _GRP * C2), 1) // C2
    return jnp.where(rows == cols, tiled, 0.0).astype(jnp.bfloat16)


def _prep_sum_body(w_ref, o_ref, *, K, C2):
    w = w_ref[...]
    wm = jnp.max(w, axis=-1, keepdims=True)
    wl = jnp.log(jnp.sum(jnp.exp(w - wm), axis=-1, keepdims=True)) + wm
    wn = jnp.exp(w - wl).reshape(_GRP * K, C2)
    o_ref[...] = _blockdiag(wn, K, C2)[None]


def _prep_leaf_body(ip_ref, o_ref, *, K, C):
    ip = ip_ref[...]
    m = jnp.max(ip, axis=-1, keepdims=True)
    lse = jnp.log(jnp.sum(jnp.exp(ip - m), axis=-1, keepdims=True)) + m
    ipn = (ip - lse).reshape(_GRP * K, C)
    o_ref[...] = _blockdiag(ipn, K, C)[None]


def _body(x_ref, wt_ref, wg_ref, wgi_ref, rp_ref, o_ref, cur_ref,
          *, V, K, C, Bt, L):

    def gather_chunk(gi, _):
        X = x_ref[pl.ds(gi * _GRP, _GRP), :]
        cc = lax.broadcasted_iota(jnp.int32, (_GRP, C, Bt), 1)
        oh = (X[:, None, :] == cc).astype(jnp.bfloat16)
        ohb = oh.reshape(_GRP * C, Bt)
        Wi = wgi_ref[gi]
        o = lax.dot_general(Wi, ohb, (((1,), (0,)), ((), ())),
                            preferred_element_type=jnp.float32)
        cur_ref[pl.ds(gi * _GRP, _GRP)] = o.reshape(_GRP, K, Bt)
        return 0
    jax.lax.fori_loop(0, V // _GRP, gather_chunk, 0)

    R = V
    goff = 0
    for _ in range(L):
        Rn = R // 2
        if Rn < _GRP:
            break

        def layer_chunk(ci, _, goff=goff):
            r0 = ci * _GRP
            p = cur_ref[pl.ds(2 * r0, 2 * _GRP)].reshape(_GRP, 2, K, Bt)
            left = p[:, 0]
            right = p[:, 1]
            ml = jnp.max(left, axis=1, keepdims=True)
            mr = jnp.max(right, axis=1, keepdims=True)
            el = jnp.exp(left - ml)
            er = jnp.exp(right - mr)
            E = jnp.concatenate(
                [el[:, i, :][:, None, :] * er for i in range(K)], axis=1)
            Eb = E.reshape(_GRP * K * K, Bt).astype(jnp.bfloat16)
            Wb = wg_ref[goff + ci]
            o = lax.dot_general(Wb, Eb, (((1,), (0,)), ((), ())),
                                preferred_element_type=jnp.float32)
            o = o.reshape(_GRP, K, Bt)
            cur_ref[pl.ds(r0, _GRP)] = jnp.log(o + 1e-38) + ml + mr
            return 0

        jax.lax.fori_loop(0, Rn // _GRP, layer_chunk, 0)
        goff += Rn // _GRP
        R = Rn

    toff = 0
    while R > 1:
        Rn = R // 2
        p = cur_ref[pl.ds(0, 2 * Rn)].reshape(Rn, 2, K, Bt)
        left = p[:, 0]
        right = p[:, 1]
        ml = jnp.max(left, axis=1, keepdims=True)
        mr = jnp.max(right, axis=1, keepdims=True)
        el = jnp.exp(left - ml)
        er = jnp.exp(right - mr)
        w = wt_ref[pl.ds(toff, Rn)]
        wm = jnp.max(w, axis=-1, keepdims=True)
        wl = jnp.log(jnp.sum(jnp.exp(w - wm), axis=-1, keepdims=True)) + wm
        Wn = jnp.exp(w - wl)
        acc = None
        for i in range(K):
            t = None
            for j in range(K):
                term = Wn[:, :, i * K + j][:, :, None] * er[:, j, :][:, None, :]
                t = term if t is None else t + term
            contrib = el[:, i, :][:, None, :] * t
            acc = contrib if acc is None else acc + contrib
        cur_ref[pl.ds(0, Rn)] = jnp.log(acc + 1e-38) + ml + mr
        toff += Rn
        R = Rn

    rp = rp_ref[...]
    rm = jnp.max(rp, axis=0, keepdims=True)
    rl = jnp.log(jnp.sum(jnp.exp(rp - rm), axis=0, keepdims=True)) + rm
    z = cur_ref[0] + (rp - rl)
    zm = jnp.max(z, axis=0, keepdims=True)
    lls = jnp.log(jnp.sum(jnp.exp(z - zm), axis=0, keepdims=True)) + zm
    o_ref[...] = lls[None]


def kernel(inputs, input_params, sum_params, root_params):
    B, V = inputs.shape
    _, K, C = input_params.shape
    C2 = K * K
    L = int(math.log2(V))
    Bt = 256
    G = B // Bt
    NG = sum(
        (V >> (l + 1)) // _GRP for l in range(L) if (V >> (l + 1)) >= _GRP)
    NGI = V // _GRP

    wg = pl.pallas_call(
        functools.partial(_prep_sum_body, K=K, C2=C2),
        grid=(NG,),
        in_specs=[pl.BlockSpec((_GRP, K, C2), lambda g: (g, 0, 0))],
        out_specs=pl.BlockSpec((1, _GRP * K, _GRP * C2), lambda g: (g, 0, 0)),
        out_shape=jax.ShapeDtypeStruct((NG, _GRP * K, _GRP * C2), jnp.bfloat16),
        compiler_params=pltpu.CompilerParams(
            dimension_semantics=("arbitrary",),
        ),
    )(sum_params)

    wgi = pl.pallas_call(
        functools.partial(_prep_leaf_body, K=K, C=C),
        grid=(NGI,),
        in_specs=[pl.BlockSpec((_GRP, K, C), lambda g: (g, 0, 0))],
        out_specs=pl.BlockSpec((1, _GRP * K, _GRP * C), lambda g: (g, 0, 0)),
        out_shape=jax.ShapeDtypeStruct((NGI, _GRP * K, _GRP * C), jnp.bfloat16),
        compiler_params=pltpu.CompilerParams(
            dimension_semantics=("arbitrary",),
        ),
    )(input_params)

    xT = inputs.T
    rpb = jnp.broadcast_to(root_params[:, None], (K, B))
    w_tail = sum_params[V - _GRP:]

    body = functools.partial(_body, V=V, K=K, C=C, Bt=Bt, L=L)
    out = pl.pallas_call(
        body,
        grid=(G,),
        in_specs=[
            pl.BlockSpec((V, Bt), lambda g: (0, g)),
            pl.BlockSpec((_GRP - 1, K, K * K), lambda g: (0, 0, 0)),
            pl.BlockSpec((NG, _GRP * K, _GRP * C2), lambda g: (0, 0, 0)),
            pl.BlockSpec((NGI, _GRP * K, _GRP * C), lambda g: (0, 0, 0)),
            pl.BlockSpec((K, Bt), lambda g: (0, g)),
        ],
        out_specs=pl.BlockSpec((1, 1, Bt), lambda g: (g, 0, 0)),
        out_shape=jax.ShapeDtypeStruct((G, 1, Bt), jnp.float32),
        scratch_shapes=[pltpu.VMEM((V, K, Bt), jnp.float32)],
        compiler_params=pltpu.CompilerParams(
            dimension_semantics=("arbitrary",),
        ),
    )(xT, w_tail, wg, wgi, rpb)
    return out.reshape(B, 1)


# speedup vs baseline: 535.7496x; 1.2727x over previous
import functools
import math

import jax
import jax.numpy as jnp
from jax import lax
from jax.experimental import pallas as pl
from jax.experimental.pallas import tpu as pltpu

_GRP = 16


def _blockdiag(wn, K, C2):
    tiled = jnp.concatenate([wn] * _GRP, axis=1)
    rows = lax.broadcasted_iota(jnp.int32, (_GRP * K, _GRP * C2), 0) // K
    cols = lax.broadcasted_iota(jnp.int32, (_GRP * K, _GRP * C2), 1) // C2
    return jnp.where(rows == cols, tiled, 0.0).astype(jnp.bfloat16)


def _body(x_ref, ip_ref, w_ref, rp_ref, o_ref, cur_ref, wg_ref, wgi_ref,
          *, V, K, C, Bt, L, NG):
    C2 = K * K

    @pl.when(pl.program_id(0) == 0)
    def _prep():
        def sum_grp(gi, _):
            w = w_ref[pl.ds(gi * _GRP, _GRP)]
            wm = jnp.max(w, axis=-1, keepdims=True)
            wl = jnp.log(jnp.sum(jnp.exp(w - wm), axis=-1, keepdims=True)) + wm
            wn = jnp.exp(w - wl).reshape(_GRP * K, C2)
            wg_ref[gi] = _blockdiag(wn, K, C2)
            return 0
        jax.lax.fori_loop(0, NG, sum_grp, 0)

        def leaf_grp(gi, _):
            ip = ip_ref[pl.ds(gi * _GRP, _GRP)]
            m = jnp.max(ip, axis=-1, keepdims=True)
            lse = jnp.log(jnp.sum(jnp.exp(ip - m), axis=-1, keepdims=True)) + m
            ipn = (ip - lse).reshape(_GRP * K, C)
            wgi_ref[gi] = _blockdiag(ipn, K, C)
            return 0
        jax.lax.fori_loop(0, V // _GRP, leaf_grp, 0)

    def gather_chunk(gi, _):
        X = x_ref[pl.ds(gi * _GRP, _GRP), :]
        cc = lax.broadcasted_iota(jnp.int32, (_GRP, C, Bt), 1)
        oh = (X[:, None, :] == cc).astype(jnp.bfloat16)
        ohb = oh.reshape(_GRP * C, Bt)
        Wi = wgi_ref[gi]
        o = lax.dot_general(Wi, ohb, (((1,), (0,)), ((), ())),
                            preferred_element_type=jnp.float32)
        cur_ref[pl.ds(gi * _GRP, _GRP)] = o.reshape(_GRP, K, Bt)
        return 0
    jax.lax.fori_loop(0, V // _GRP, gather_chunk, 0)

    R = V
    goff = 0
    for _ in range(L):
        Rn = R // 2
        if Rn < _GRP:
            break

        def layer_chunk(ci, _, goff=goff):
            r0 = ci * _GRP
            p = cur_ref[pl.ds(2 * r0, 2 * _GRP)].reshape(_GRP, 2, K, Bt)
            left = p[:, 0]
            right = p[:, 1]
            ml = jnp.max(left, axis=1, keepdims=True)
            mr = jnp.max(right, axis=1, keepdims=True)
            el = jnp.exp(left - ml)
            er = jnp.exp(right - mr)
            E = jnp.concatenate(
                [el[:, i, :][:, None, :] * er for i in range(K)], axis=1)
            Eb = E.reshape(_GRP * K * K, Bt).astype(jnp.bfloat16)
            Wb = wg_ref[goff + ci]
            o = lax.dot_general(Wb, Eb, (((1,), (0,)), ((), ())),
                                preferred_element_type=jnp.float32)
            o = o.reshape(_GRP, K, Bt)
            cur_ref[pl.ds(r0, _GRP)] = jnp.log(o + 1e-38) + ml + mr
            return 0

        jax.lax.fori_loop(0, Rn // _GRP, layer_chunk, 0)
        goff += Rn // _GRP
        R = Rn

    off = V - R
    while R > 1:
        Rn = R // 2
        p = cur_ref[pl.ds(0, 2 * Rn)].reshape(Rn, 2, K, Bt)
        left = p[:, 0]
        right = p[:, 1]
        ml = jnp.max(left, axis=1, keepdims=True)
        mr = jnp.max(right, axis=1, keepdims=True)
        el = jnp.exp(left - ml)
        er = jnp.exp(right - mr)
        w = w_ref[pl.ds(off, Rn)]
        wm = jnp.max(w, axis=-1, keepdims=True)
        wl = jnp.log(jnp.sum(jnp.exp(w - wm), axis=-1, keepdims=True)) + wm
        Wn = jnp.exp(w - wl)
        acc = None
        for i in range(K):
            t = None
            for j in range(K):
                term = Wn[:, :, i * K + j][:, :, None] * er[:, j, :][:, None, :]
                t = term if t is None else t + term
            contrib = el[:, i, :][:, None, :] * t
            acc = contrib if acc is None else acc + contrib
        cur_ref[pl.ds(0, Rn)] = jnp.log(acc + 1e-38) + ml + mr
        off += Rn
        R = Rn

    rp = rp_ref[...]
    rm = jnp.max(rp, axis=0, keepdims=True)
    rl = jnp.log(jnp.sum(jnp.exp(rp - rm), axis=0, keepdims=True)) + rm
    z = cur_ref[0] + (rp - rl)
    zm = jnp.max(z, axis=0, keepdims=True)
    lls = jnp.log(jnp.sum(jnp.exp(z - zm), axis=0, keepdims=True)) + zm
    o_ref[...] = lls[None]


def kernel(inputs, input_params, sum_params, root_params):
    B, V = inputs.shape
    _, K, C = input_params.shape
    C2 = K * K
    L = int(math.log2(V))
    Bt = 256
    G = B // Bt
    NG = sum(
        (V >> (l + 1)) // _GRP for l in range(L) if (V >> (l + 1)) >= _GRP)
    NGI = V // _GRP

    xT = inputs.T
    rpb = jnp.broadcast_to(root_params[:, None], (K, B))

    body = functools.partial(_body, V=V, K=K, C=C, Bt=Bt, L=L, NG=NG)
    out = pl.pallas_call(
        body,
        grid=(G,),
        in_specs=[
            pl.BlockSpec((V, Bt), lambda g: (0, g)),
            pl.BlockSpec((V, K, C), lambda g: (0, 0, 0)),
            pl.BlockSpec((V - 1, K, K * K), lambda g: (0, 0, 0)),
            pl.BlockSpec((K, Bt), lambda g: (0, g)),
        ],
        out_specs=pl.BlockSpec((1, 1, Bt), lambda g: (g, 0, 0)),
        out_shape=jax.ShapeDtypeStruct((G, 1, Bt), jnp.float32),
        scratch_shapes=[
            pltpu.VMEM((V, K, Bt), jnp.float32),
            pltpu.VMEM((NG, _GRP * K, _GRP * C2), jnp.bfloat16),
            pltpu.VMEM((NGI, _GRP * K, _GRP * C), jnp.bfloat16),
        ],
        compiler_params=pltpu.CompilerParams(
            dimension_semantics=("arbitrary",),
        ),
    )(xT, input_params, sum_params, rpb)
    return out.reshape(B, 1)


# speedup vs baseline: 624.4159x; 1.1655x over previous
import functools
import math

import jax
import jax.numpy as jnp
from jax import lax
from jax.experimental import pallas as pl
from jax.experimental.pallas import tpu as pltpu

_GRP = 16


def _blockdiag(wn, K, C2):
    tiled = jnp.concatenate([wn] * _GRP, axis=1)
    rows = lax.broadcasted_iota(jnp.int32, (_GRP * K, _GRP * C2), 0) // K
    cols = lax.broadcasted_iota(jnp.int32, (_GRP * K, _GRP * C2), 1) // C2
    return jnp.where(rows == cols, tiled, 0.0).astype(jnp.bfloat16)


def _body(x_ref, ip_ref, w_ref, rp_ref, o_ref, cur_ref, wg_ref, wgi_ref,
          *, V, K, C, Bt, L, NG):
    C2 = K * K

    @pl.when(pl.program_id(0) == 0)
    def _prep():
        def sum_grp(gi, _):
            w = w_ref[pl.ds(gi * _GRP, _GRP)]
            wm = jnp.max(w, axis=-1, keepdims=True)
            wl = jnp.log(jnp.sum(jnp.exp(w - wm), axis=-1, keepdims=True)) + wm
            wn = jnp.exp(w - wl).reshape(_GRP * K, C2)
            wg_ref[gi] = _blockdiag(wn, K, C2)
            return 0
        jax.lax.fori_loop(0, NG, sum_grp, 0)

        def leaf_grp(gi, _):
            ip = ip_ref[pl.ds(gi * _GRP, _GRP)]
            m = jnp.max(ip, axis=-1, keepdims=True)
            lse = jnp.log(jnp.sum(jnp.exp(ip - m), axis=-1, keepdims=True)) + m
            ipn = (ip - lse).reshape(_GRP * K, C)
            wgi_ref[gi] = _blockdiag(ipn, K, C)
            return 0
        jax.lax.fori_loop(0, V // _GRP, leaf_grp, 0)

    def gather_chunk(gi, _):
        X = x_ref[pl.ds(gi * _GRP, _GRP), :]
        cc = lax.broadcasted_iota(jnp.int32, (_GRP, C, Bt), 1)
        oh = (X[:, None, :] == cc).astype(jnp.bfloat16)
        ohb = oh.reshape(_GRP * C, Bt)
        Wi = wgi_ref[gi]
        o = lax.dot_general(Wi, ohb, (((1,), (0,)), ((), ())),
                            preferred_element_type=jnp.float32)
        cur_ref[pl.ds(gi * _GRP, _GRP)] = o.reshape(_GRP, K, Bt)
        return 0
    jax.lax.fori_loop(0, V // _GRP, gather_chunk, 0, unroll=2)

    R = V
    goff = 0
    for _ in range(L):
        Rn = R // 2
        if Rn < _GRP:
            break

        def layer_chunk(ci, _, goff=goff):
            r0 = ci * _GRP
            p = cur_ref[pl.ds(2 * r0, 2 * _GRP)].reshape(_GRP, 2, K, Bt)
            left = p[:, 0]
            right = p[:, 1]
            ml = jnp.max(left, axis=1, keepdims=True)
            mr = jnp.max(right, axis=1, keepdims=True)
            el = jnp.exp(left - ml)
            er = jnp.exp(right - mr)
            E = jnp.concatenate(
                [el[:, i, :][:, None, :] * er for i in range(K)], axis=1)
            Eb = E.reshape(_GRP * K * K, Bt).astype(jnp.bfloat16)
            Wb = wg_ref[goff + ci]
            o = lax.dot_general(Wb, Eb, (((1,), (0,)), ((), ())),
                                preferred_element_type=jnp.float32)
            o = o.reshape(_GRP, K, Bt)
            cur_ref[pl.ds(r0, _GRP)] = jnp.log(o + 1e-38) + ml + mr
            return 0

        jax.lax.fori_loop(0, Rn // _GRP, layer_chunk, 0,
                          unroll=2 if Rn // _GRP >= 2 else 1)
        goff += Rn // _GRP
        R = Rn

    off = V - R
    while R > 1:
        Rn = R // 2
        p = cur_ref[pl.ds(0, 2 * Rn)].reshape(Rn, 2, K, Bt)
        left = p[:, 0]
        right = p[:, 1]
        ml = jnp.max(left, axis=1, keepdims=True)
        mr = jnp.max(right, axis=1, keepdims=True)
        el = jnp.exp(left - ml)
        er = jnp.exp(right - mr)
        w = w_ref[pl.ds(off, Rn)]
        wm = jnp.max(w, axis=-1, keepdims=True)
        wl = jnp.log(jnp.sum(jnp.exp(w - wm), axis=-1, keepdims=True)) + wm
        Wn = jnp.exp(w - wl)
        acc = None
        for i in range(K):
            t = None
            for j in range(K):
                term = Wn[:, :, i * K + j][:, :, None] * er[:, j, :][:, None, :]
                t = term if t is None else t + term
            contrib = el[:, i, :][:, None, :] * t
            acc = contrib if acc is None else acc + contrib
        cur_ref[pl.ds(0, Rn)] = jnp.log(acc + 1e-38) + ml + mr
        off += Rn
        R = Rn

    rp = rp_ref[...]
    rm = jnp.max(rp, axis=0, keepdims=True)
    rl = jnp.log(jnp.sum(jnp.exp(rp - rm), axis=0, keepdims=True)) + rm
    z = cur_ref[0] + (rp - rl)
    zm = jnp.max(z, axis=0, keepdims=True)
    lls = jnp.log(jnp.sum(jnp.exp(z - zm), axis=0, keepdims=True)) + zm
    o_ref[...] = lls[None]


def kernel(inputs, input_params, sum_params, root_params):
    B, V = inputs.shape
    _, K, C = input_params.shape
    C2 = K * K
    L = int(math.log2(V))
    Bt = 256
    G = B // Bt
    NG = sum(
        (V >> (l + 1)) // _GRP for l in range(L) if (V >> (l + 1)) >= _GRP)
    NGI = V // _GRP

    xT = inputs.T
    rpb = jnp.broadcast_to(root_params[:, None], (K, B))

    body = functools.partial(_body, V=V, K=K, C=C, Bt=Bt, L=L, NG=NG)
    out = pl.pallas_call(
        body,
        grid=(G,),
        in_specs=[
            pl.BlockSpec((V, Bt), lambda g: (0, g)),
            pl.BlockSpec((V, K, C), lambda g: (0, 0, 0)),
            pl.BlockSpec((V - 1, K, K * K), lambda g: (0, 0, 0)),
            pl.BlockSpec((K, Bt), lambda g: (0, g)),
        ],
        out_specs=pl.BlockSpec((1, 1, Bt), lambda g: (g, 0, 0)),
        out_shape=jax.ShapeDtypeStruct((G, 1, Bt), jnp.float32),
        scratch_shapes=[
            pltpu.VMEM((V, K, Bt), jnp.float32),
            pltpu.VMEM((NG, _GRP * K, _GRP * C2), jnp.bfloat16),
            pltpu.VMEM((NGI, _GRP * K, _GRP * C), jnp.bfloat16),
        ],
        compiler_params=pltpu.CompilerParams(
            dimension_semantics=("arbitrary",),
        ),
    )(xT, input_params, sum_params, rpb)
    return out.reshape(B, 1)


# speedup vs baseline: 676.8214x; 1.0839x over previous
import functools
import math

import jax
import jax.numpy as jnp
from jax import lax
from jax.experimental import pallas as pl
from jax.experimental.pallas import tpu as pltpu

_GRP = 16


def _blockdiag(wn, K, C2):
    tiled = jnp.concatenate([wn] * _GRP, axis=1)
    rows = lax.broadcasted_iota(jnp.int32, (_GRP * K, _GRP * C2), 0) // K
    cols = lax.broadcasted_iota(jnp.int32, (_GRP * K, _GRP * C2), 1) // C2
    return jnp.where(rows == cols, tiled, 0.0).astype(jnp.bfloat16)


def _body(x_ref, ip_ref, w_ref, rp_ref, o_ref, cur_ref, wg_ref, wgi_ref,
          *, V, K, C, Bt, L, NG):
    C2 = K * K

    @pl.when(pl.program_id(0) == 0)
    def _prep():
        def sum_grp(gi, _):
            w = w_ref[pl.ds(gi * _GRP, _GRP)]
            wm = jnp.max(w, axis=-1, keepdims=True)
            wl = jnp.log(jnp.sum(jnp.exp(w - wm), axis=-1, keepdims=True)) + wm
            wn = jnp.exp(w - wl).reshape(_GRP * K, C2)
            wg_ref[gi] = _blockdiag(wn, K, C2)
            return 0
        jax.lax.fori_loop(0, NG, sum_grp, 0)

        def leaf_grp(gi, _):
            ip = ip_ref[pl.ds(gi * _GRP, _GRP)]
            m = jnp.max(ip, axis=-1, keepdims=True)
            lse = jnp.log(jnp.sum(jnp.exp(ip - m), axis=-1, keepdims=True)) + m
            ipn = (ip - lse).reshape(_GRP * K, C)
            wgi_ref[gi] = _blockdiag(ipn, K, C)
            return 0
        jax.lax.fori_loop(0, V // _GRP, leaf_grp, 0)

    def gather_chunk(gi, _):
        X = x_ref[pl.ds(gi * _GRP, _GRP), :]
        cc = lax.broadcasted_iota(jnp.int32, (_GRP, C, Bt), 1)
        oh = (X[:, None, :] == cc).astype(jnp.bfloat16)
        ohb = oh.reshape(_GRP * C, Bt)
        Wi = wgi_ref[gi]
        o = lax.dot_general(Wi, ohb, (((1,), (0,)), ((), ())),
                            preferred_element_type=jnp.float32)
        cur_ref[pl.ds(gi * _GRP, _GRP)] = o.reshape(_GRP, K, Bt)
        return 0
    jax.lax.fori_loop(0, V // _GRP, gather_chunk, 0, unroll=4)

    R = V
    goff = 0
    for _ in range(L):
        Rn = R // 2
        if Rn < _GRP:
            break

        def layer_chunk(ci, _, goff=goff):
            r0 = ci * _GRP
            p = cur_ref[pl.ds(2 * r0, 2 * _GRP)].reshape(_GRP, 2, K, Bt)
            left = p[:, 0]
            right = p[:, 1]
            ml = jnp.max(left, axis=1, keepdims=True)
            mr = jnp.max(right, axis=1, keepdims=True)
            el = jnp.exp(left - ml)
            er = jnp.exp(right - mr)
            E = jnp.concatenate(
                [el[:, i, :][:, None, :] * er for i in range(K)], axis=1)
            Eb = E.reshape(_GRP * K * K, Bt).astype(jnp.bfloat16)
            Wb = wg_ref[goff + ci]
            o = lax.dot_general(Wb, Eb, (((1,), (0,)), ((), ())),
                                preferred_element_type=jnp.float32)
            o = o.reshape(_GRP, K, Bt)
            cur_ref[pl.ds(r0, _GRP)] = jnp.log(o + 1e-38) + ml + mr
            return 0

        jax.lax.fori_loop(0, Rn // _GRP, layer_chunk, 0,
                          unroll=min(4, Rn // _GRP))
        goff += Rn // _GRP
        R = Rn

    off = V - R
    while R > 1:
        Rn = R // 2
        p = cur_ref[pl.ds(0, 2 * Rn)].reshape(Rn, 2, K, Bt)
        left = p[:, 0]
        right = p[:, 1]
        ml = jnp.max(left, axis=1, keepdims=True)
        mr = jnp.max(right, axis=1, keepdims=True)
        el = jnp.exp(left - ml)
        er = jnp.exp(right - mr)
        w = w_ref[pl.ds(off, Rn)]
        wm = jnp.max(w, axis=-1, keepdims=True)
        wl = jnp.log(jnp.sum(jnp.exp(w - wm), axis=-1, keepdims=True)) + wm
        Wn = jnp.exp(w - wl)
        acc = None
        for i in range(K):
            t = None
            for j in range(K):
                term = Wn[:, :, i * K + j][:, :, None] * er[:, j, :][:, None, :]
                t = term if t is None else t + term
            contrib = el[:, i, :][:, None, :] * t
            acc = contrib if acc is None else acc + contrib
        cur_ref[pl.ds(0, Rn)] = jnp.log(acc + 1e-38) + ml + mr
        off += Rn
        R = Rn

    rp = rp_ref[...]
    rm = jnp.max(rp, axis=0, keepdims=True)
    rl = jnp.log(jnp.sum(jnp.exp(rp - rm), axis=0, keepdims=True)) + rm
    z = cur_ref[0] + (rp - rl)
    zm = jnp.max(z, axis=0, keepdims=True)
    lls = jnp.log(jnp.sum(jnp.exp(z - zm), axis=0, keepdims=True)) + zm
    o_ref[...] = lls[None]


def kernel(inputs, input_params, sum_params, root_params):
    B, V = inputs.shape
    _, K, C = input_params.shape
    C2 = K * K
    L = int(math.log2(V))
    Bt = 256
    G = B // Bt
    NG = sum(
        (V >> (l + 1)) // _GRP for l in range(L) if (V >> (l + 1)) >= _GRP)
    NGI = V // _GRP

    xT = inputs.T
    rpb = jnp.broadcast_to(root_params[:, None], (K, B))

    body = functools.partial(_body, V=V, K=K, C=C, Bt=Bt, L=L, NG=NG)
    out = pl.pallas_call(
        body,
        grid=(G,),
        in_specs=[
            pl.BlockSpec((V, Bt), lambda g: (0, g)),
            pl.BlockSpec((V, K, C), lambda g: (0, 0, 0)),
            pl.BlockSpec((V - 1, K, K * K), lambda g: (0, 0, 0)),
            pl.BlockSpec((K, Bt), lambda g: (0, g)),
        ],
        out_specs=pl.BlockSpec((1, 1, Bt), lambda g: (g, 0, 0)),
        out_shape=jax.ShapeDtypeStruct((G, 1, Bt), jnp.float32),
        scratch_shapes=[
            pltpu.VMEM((V, K, Bt), jnp.float32),
            pltpu.VMEM((NG, _GRP * K, _GRP * C2), jnp.bfloat16),
            pltpu.VMEM((NGI, _GRP * K, _GRP * C), jnp.bfloat16),
        ],
        compiler_params=pltpu.CompilerParams(
            dimension_semantics=("arbitrary",),
        ),
    )(xT, input_params, sum_params, rpb)
    return out.reshape(B, 1)
